# baseline (speedup 1.0000x reference)
import functools

import jax
import jax.numpy as jnp
import numpy as np
from jax.experimental import pallas as pl
from jax.experimental.pallas import tpu as pltpu

_C_BLK = 32


def _hat_integral_cdf(t):
    t = np.asarray(t, np.float32)
    left = 0.5 * (t + 1.0) ** 2
    right = 1.0 - 0.5 * (1.0 - t) ** 2
    return np.where(t <= -1.0, 0.0,
           np.where(t <= 0.0, left,
           np.where(t <= 1.0, right, 1.0))).astype(np.float32)


def _axis_weights(n_out, extent, n_in):
    bin_sz = extent / float(n_out)
    p = np.arange(n_out, dtype=np.float32)[:, None]
    g = np.arange(n_in, dtype=np.float32)[None, :]
    w = _hat_integral_cdf((p + 1.0) * bin_sz - g) - _hat_integral_cdf(p * bin_sz - g)
    return w / bin_sz if bin_sz > 0.0 else np.zeros_like(w)


@functools.lru_cache(maxsize=None)
def _kron_weights_bf16(h1, w1, h2, w2):
    wy = _axis_weights(h2, float(w1 - 1), h1)
    wx = _axis_weights(w2, float(h1 - 1), w1)
    wk = np.einsum('ph,qw->pqhw', wy, wx).reshape(h2 * w2, h1 * w1)
    return np.asarray(wk, dtype=jnp.bfloat16)


def _resize_mm(x_ref, w_ref, o_ref):
    wk = w_ref[...]
    for i in range(x_ref.shape[0]):
        o_ref[i] = jnp.dot(
            wk, x_ref[i].astype(jnp.bfloat16),
            preferred_element_type=jnp.float32,
        )


def kernel(x):
    n, c, h1, w1 = x.shape
    h2, w2 = 16, 16
    k = h1 * w1
    n_out = h2 * w2

    wk = _kron_weights_bf16(h1, w1, h2, w2)

    xt = x.transpose(1, 2, 3, 0).reshape(c, k, n)

    grid = (c // _C_BLK,)

    cost = pl.CostEstimate(
        flops=int(2 * c * n_out * k * n),
        transcendentals=0,
        bytes_accessed=int(c * k * n * 4 + n_out * k * 2 + c * n_out * n * 4),
    )

    out = pl.pallas_call(
        _resize_mm,
        out_shape=jax.ShapeDtypeStruct((c, n_out, n), jnp.float32),
        grid=grid,
        in_specs=[
            pl.BlockSpec((_C_BLK, k, n), lambda i: (i, 0, 0)),
            pl.BlockSpec((n_out, k), lambda i: (0, 0)),
        ],
        out_specs=pl.BlockSpec((_C_BLK, n_out, n), lambda i: (i, 0, 0)),
        compiler_params=pltpu.CompilerParams(
            dimension_semantics=("parallel",),
            vmem_limit_bytes=50 * 1024 * 1024,
        ),
        cost_estimate=cost,
    )(xt, wk)

    return jnp.squeeze(out.reshape(c, h2, w2, n).transpose(3, 0, 1, 2))


# speedup vs baseline: 1.0084x; 1.0084x over previous
import functools

import jax
import jax.numpy as jnp
import numpy as np
from jax.experimental import pallas as pl
from jax.experimental.pallas import tpu as pltpu

_C_BLK = 16


def _hat_integral_cdf(t):
    t = np.asarray(t, np.float32)
    left = 0.5 * (t + 1.0) ** 2
    right = 1.0 - 0.5 * (1.0 - t) ** 2
    return np.where(t <= -1.0, 0.0,
           np.where(t <= 0.0, left,
           np.where(t <= 1.0, right, 1.0))).astype(np.float32)


def _axis_weights(n_out, extent, n_in):
    bin_sz = extent / float(n_out)
    p = np.arange(n_out, dtype=np.float32)[:, None]
    g = np.arange(n_in, dtype=np.float32)[None, :]
    w = _hat_integral_cdf((p + 1.0) * bin_sz - g) - _hat_integral_cdf(p * bin_sz - g)
    return w / bin_sz if bin_sz > 0.0 else np.zeros_like(w)


@functools.lru_cache(maxsize=None)
def _kron_weights_bf16(h1, w1, h2, w2):
    wy = _axis_weights(h2, float(w1 - 1), h1)
    wx = _axis_weights(w2, float(h1 - 1), w1)
    wk = np.einsum('ph,qw->pqhw', wy, wx).reshape(h2 * w2, h1 * w1)
    return np.asarray(wk, dtype=jnp.bfloat16)


def _resize_mm(x_ref, w_ref, o_ref):
    wk = w_ref[...]
    for i in range(x_ref.shape[0]):
        o_ref[i] = jnp.dot(
            wk, x_ref[i].astype(jnp.bfloat16),
            preferred_element_type=jnp.float32,
        )


def kernel(x):
    n, c, h1, w1 = x.shape
    h2, w2 = 16, 16
    k = h1 * w1
    n_out = h2 * w2

    wk = _kron_weights_bf16(h1, w1, h2, w2)

    xt = x.transpose(1, 2, 3, 0).reshape(c, k, n)

    grid = (c // _C_BLK,)

    cost = pl.CostEstimate(
        flops=int(2 * c * n_out * k * n),
        transcendentals=0,
        bytes_accessed=int(c * k * n * 4 + n_out * k * 2 + c * n_out * n * 4),
    )

    out = pl.pallas_call(
        _resize_mm,
        out_shape=jax.ShapeDtypeStruct((c, n_out, n), jnp.float32),
        grid=grid,
        in_specs=[
            pl.BlockSpec((_C_BLK, k, n), lambda i: (i, 0, 0)),
            pl.BlockSpec(memory_space=pltpu.MemorySpace.VMEM),
        ],
        out_specs=pl.BlockSpec((_C_BLK, n_out, n), lambda i: (i, 0, 0)),
        compiler_params=pltpu.CompilerParams(
            dimension_semantics=("parallel",),
            vmem_limit_bytes=40 * 1024 * 1024,
        ),
        cost_estimate=cost,
    )(xt, wk)

    return jnp.squeeze(out.reshape(c, h2, w2, n).transpose(3, 0, 1, 2))
